# baseline (speedup 1.0000x reference)
import numpy as np
import jax
import jax.numpy as jnp
from jax import lax
from jax.experimental import pallas as pl
from jax.experimental.pallas import tpu as pltpu
from jax.experimental.pallas import tpu_sc as plsc

_OUT_CH = 192
_FEAT_DIM = _OUT_CH // 6
_NW = 32
_UNITS = 96
_ROWS = 1024
_NS = 32
_CH = 256

_PIO2_HI = np.float32(1.57079637050628662109375)
_PIO2_LO = np.float32(-4.37113900018624283e-8)
_MAGIC = np.float32(1.5 * 2.0**23)
_S1 = np.float32(-1.6666654611e-1)
_S2 = np.float32(8.3321608736e-3)
_S3 = np.float32(-1.9515295891e-4)
_C0 = np.float32(2.443315711809948e-5)
_C1 = np.float32(-1.388731625493765e-3)
_C2 = np.float32(4.166664568298827e-2)


def _fast_sin(t):
    kb = t * np.float32(0.6366197723675814) + _MAGIC
    bits = lax.bitcast_convert_type(kb, jnp.int32) - np.int32(0x4B400000)
    k = bits.astype(jnp.float32)
    y = t - k * _PIO2_HI
    y = y - k * _PIO2_LO
    z = y * y
    ps = _S3 * z + _S2
    ps = ps * z + _S1
    s = y + (y * z) * ps
    pc = _C0 * z + _C1
    pc = pc * z + _C2
    c = (z * z) * pc + (np.float32(1.0) - np.float32(0.5) * z)
    swap = (bits & 1) == 1
    base = jnp.where(swap, c, s)
    flip = (bits & 2) << 30
    return lax.bitcast_convert_type(
        lax.bitcast_convert_type(base, jnp.int32) ^ flip, jnp.float32)


def _body(r_hbm, x_hbm, scab_hbm, phab_hbm, out_hbm,
          sc_v, ph_v, r_v, x_v, o_v):
    wid = lax.axis_index("s") * 2 + lax.axis_index("c")

    @pl.loop(0, 3)
    def unit_loop(u_k):
        u = wid * 3 + u_k
        b = u // 12
        rem = u - 12 * b
        i = rem // 4
        cq = rem - 4 * i
        c0 = i * 64 + cq * 16

        @pl.loop(0, _ROWS // _CH)
        def chunk_loop(ch):
            pltpu.sync_copy(r_hbm.at[b, i, pl.ds(ch * _CH, _CH), :], r_v)

            @pl.loop(0, 16)
            def plane_loop(cc):
                c = c0 + cc
                pltpu.sync_copy(scab_hbm.at[c], sc_v)
                pltpu.sync_copy(phab_hbm.at[c], ph_v)
                pltpu.sync_copy(x_hbm.at[b, c, pl.ds(ch * _CH, _CH), :], x_v)
                s_vec = sc_v[...]
                p_vec = ph_v[...]

                @pl.loop(0, _CH // 4)
                def row_loop(rq):
                    for uu in range(4):
                        rw = rq * 4 + uu
                        for half in range(2):
                            sl = pl.ds(16 * half, 16)
                            t = r_v[rw, sl] * s_vec + p_vec
                            pe = _fast_sin(t)
                            o_v[rw, sl] = x_v[rw, sl] * pe + pe

                pltpu.sync_copy(o_v, out_hbm.at[b, c, pl.ds(ch * _CH, _CH), :])


def kernel(re_xyz, x):
    B, _, npoint, nsample = re_xyz.shape
    C = x.shape[1]

    fr = np.arange(_FEAT_DIM, dtype=np.float32)
    dim_mat = np.power(np.float32(500.0), (np.float32(1.0 / _FEAT_DIM) * fr),
                       dtype=np.float32)
    scale_f = (np.float32(50.0) / dim_mat).astype(np.float32)
    sc_c = np.zeros((C,), np.float32)
    ph_c = np.zeros((C,), np.float32)
    for c in range(C):
        j = c % 64
        f = j % _FEAT_DIM
        sc_c[c] = scale_f[f]
        ph_c[c] = 0.0 if j < _FEAT_DIM else np.float32(np.pi / 2)
    scab = jnp.asarray(np.repeat(sc_c[:, None], 16, axis=1))
    phab = jnp.asarray(np.repeat(ph_c[:, None], 16, axis=1))

    mesh = plsc.VectorSubcoreMesh(core_axis_name="c", subcore_axis_name="s")
    run = pl.kernel(
        _body,
        out_type=jax.ShapeDtypeStruct((B, C, npoint, nsample), jnp.float32),
        mesh=mesh,
        scratch_types=[
            pltpu.VMEM((16,), jnp.float32),
            pltpu.VMEM((16,), jnp.float32),
            pltpu.VMEM((_CH, _NS), jnp.float32),
            pltpu.VMEM((_CH, _NS), jnp.float32),
            pltpu.VMEM((_CH, _NS), jnp.float32),
        ],
    )
    return run(re_xyz, x, scab, phab)


# speedup vs baseline: 2.8870x; 2.8870x over previous
import numpy as np
import jax
import jax.numpy as jnp
from jax.experimental import pallas as pl

_OUT_CH = 192
_FEAT_DIM = _OUT_CH // 6
_BN = 4096

_TWO_OVER_PI = 0.6366197723675814
_PIO2_HI = np.float32(1.57079637050628662109375)
_PIO2_LO = np.float32(-4.37113900018624283e-8)
_MAGIC = np.float32(1.5 * 2.0**23)

_S1 = np.float32(-1.6666654611e-1)
_S2 = np.float32(8.3321608736e-3)
_S3 = np.float32(-1.9515295891e-4)
_C0 = np.float32(2.443315711809948e-5)
_C1 = np.float32(-1.388731625493765e-3)
_C2 = np.float32(4.166664568298827e-2)


def _sincos(t):
    kb = t * np.float32(_TWO_OVER_PI) + _MAGIC
    bits = jax.lax.bitcast_convert_type(kb, jnp.int32) - np.int32(0x4B400000)
    k = bits.astype(jnp.float32)
    y = t - k * _PIO2_HI
    y = y - k * _PIO2_LO
    z = y * y
    ps = _S3 * z + _S2
    ps = ps * z + _S1
    s = y + (y * z) * ps
    pc = _C0 * z + _C1
    pc = pc * z + _C2
    c = (z * z) * pc + (np.float32(1.0) - np.float32(0.5) * z)
    swap = (bits & 1) == 1
    sin_base = jnp.where(swap, c, s)
    cos_base = jnp.where(swap, s, c)
    sin_flip = (bits & 2) << 30
    cos_flip = ((bits + 1) & 2) << 30
    sin_t = jax.lax.bitcast_convert_type(
        jax.lax.bitcast_convert_type(sin_base, jnp.int32) ^ sin_flip, jnp.float32)
    cos_t = jax.lax.bitcast_convert_type(
        jax.lax.bitcast_convert_type(cos_base, jnp.int32) ^ cos_flip, jnp.float32)
    return sin_t, cos_t


def _pe_kernel(s_ref, r_ref, o_ref):
    s = s_ref[...]
    fd = _FEAT_DIM
    for i in range(3):
        t = r_ref[:, i : i + 1, :] * s
        sin_t, cos_t = _sincos(t)
        o_ref[:, 2 * i * fd : (2 * i + 1) * fd, :] = sin_t
        o_ref[:, (2 * i + 1) * fd : (2 * i + 2) * fd, :] = cos_t


def kernel(re_xyz, x):
    B, _, npoint, nsample = re_xyz.shape
    C = x.shape[1]
    N = npoint * nsample
    r = re_xyz.reshape(B, 3, N)

    fr = jnp.arange(_FEAT_DIM, dtype=jnp.float32)
    dim_mat = jnp.power(jnp.float32(500.0), (1.0 / _FEAT_DIM) * fr)
    scale = (50.0 / dim_mat).reshape(1, _FEAT_DIM, 1)

    nblk = N // _BN
    pe = pl.pallas_call(
        _pe_kernel,
        grid=(B, nblk),
        in_specs=[
            pl.BlockSpec((1, _FEAT_DIM, 1), lambda b, n: (0, 0, 0)),
            pl.BlockSpec((1, 3, _BN), lambda b, n: (b, 0, n)),
        ],
        out_specs=pl.BlockSpec((1, C, _BN), lambda b, n: (b, 0, n)),
        out_shape=jax.ShapeDtypeStruct((B, C, N), jnp.float32),
    )(scale, r)
    pe4 = pe.reshape(B, C, npoint, nsample)
    return x * pe4 + pe4
